# baseline (speedup 1.0000x reference)
import functools

import jax
import jax.numpy as jnp
from jax import lax
from jax.experimental import pallas as pl
from jax.experimental.pallas import tpu as pltpu
from jax.experimental.pallas import tpu_sc as plsc

B, C_IN, N, K, GR = 8, 128, 4096, 16, 64
NK = N * K
M1 = B * N
M2 = B * N * K
EPS = 1e-5


def _a_body(x_ref, s1_ref, S1_ref):
    b = pl.program_id(0)
    t = pl.program_id(1)
    x = x_ref[0]

    @pl.when(jnp.logical_and(b == 0, t == 0))
    def _():
        s1_ref[...] = jnp.zeros_like(s1_ref)
        S1_ref[...] = jnp.zeros_like(S1_ref)

    s1_ref[...] += x.sum(axis=1).reshape(1, C_IN)
    S1_ref[...] += lax.dot_general(x, x, (((1,), (1,)), ((), ())),
                                   preferred_element_type=jnp.float32)


def _feat_moments(feats):
    TN = 512
    return pl.pallas_call(
        _a_body,
        grid=(B, N // TN),
        in_specs=[pl.BlockSpec((1, C_IN, TN), lambda b, t: (b, 0, t))],
        out_specs=[pl.BlockSpec((1, C_IN), lambda b, t: (0, 0)),
                   pl.BlockSpec((C_IN, C_IN), lambda b, t: (0, 0))],
        out_shape=[jax.ShapeDtypeStruct((1, C_IN), jnp.float32),
                   jax.ShapeDtypeStruct((C_IN, C_IN), jnp.float32)],
    )(feats)


def _c_body(x_ref, cnt_ref, d_ref, W1f_ref, c1f_ref, Wf_ref, bf_ref,
            T_ref, S3Q3_ref, D1_ref, D2_ref):
    b = pl.program_id(0)
    t = pl.program_id(1)

    @pl.when(jnp.logical_and(b == 0, t == 0))
    def _():
        S3Q3_ref[...] = jnp.zeros_like(S3Q3_ref)
        D1_ref[...] = jnp.zeros_like(D1_ref)
        D2_ref[...] = jnp.zeros_like(D2_ref)

    x = x_ref[0]
    nf = jax.nn.relu(
        lax.dot_general(x, W1f_ref[...], (((0,), (1,)), ((), ())),
                        preferred_element_type=jnp.float32)
        + c1f_ref[...])
    h = lax.dot_general(nf, Wf_ref[...], (((1,), (1,)), ((), ())),
                        preferred_element_type=jnp.float32) + bf_ref[...]
    T_ref[0] = jnp.concatenate(
        [h, jnp.zeros_like(h)], axis=1)

    cnt = cnt_ref[0].sum(axis=0).reshape(-1, 1)
    hw = h * cnt
    S3 = hw.sum(axis=0).reshape(1, GR)
    Q3 = (hw * h).sum(axis=0).reshape(1, GR)
    S3Q3_ref[...] += jnp.concatenate([S3, Q3], axis=0)

    d = d_ref[0]
    D1_ref[...] += d.sum(axis=0).reshape(1, 4)
    D2_ref[...] += lax.dot_general(d, d, (((0,), (0,)), ((), ())),
                                   preferred_element_type=jnp.float32)


def _table_and_stats(feats, counts4, d_all, W1f, c1f, W_feats, b_feats):
    TN = 512
    return pl.pallas_call(
        _c_body,
        grid=(B, N // TN),
        in_specs=[
            pl.BlockSpec((1, C_IN, TN), lambda b, t: (b, 0, t)),
            pl.BlockSpec((1, 4, TN), lambda b, t: (b, 0, t)),
            pl.BlockSpec((1, TN * K, 4), lambda b, t: (b, t, 0)),
            pl.BlockSpec((C_IN, C_IN), lambda b, t: (0, 0)),
            pl.BlockSpec((1, C_IN), lambda b, t: (0, 0)),
            pl.BlockSpec((GR, C_IN), lambda b, t: (0, 0)),
            pl.BlockSpec((1, GR), lambda b, t: (0, 0)),
        ],
        out_specs=[
            pl.BlockSpec((1, TN, 2 * GR), lambda b, t: (b, t, 0)),
            pl.BlockSpec((2, GR), lambda b, t: (0, 0)),
            pl.BlockSpec((1, 4), lambda b, t: (0, 0)),
            pl.BlockSpec((4, 4), lambda b, t: (0, 0)),
        ],
        out_shape=[
            jax.ShapeDtypeStruct((B, N, 2 * GR), jnp.float32),
            jax.ShapeDtypeStruct((2, GR), jnp.float32),
            jax.ShapeDtypeStruct((1, 4), jnp.float32),
            jax.ShapeDtypeStruct((4, 4), jnp.float32),
        ],
    )(feats, counts4, d_all, W1f, c1f, W_feats, b_feats)


def _d_body(is_final, G_ref, d_ref, a3_ref, c3_ref, A2_ref, c2_ref,
            Wp_ref, bp_ref, a4_ref, c4_ref, *refs):
    if is_final:
        out_refs = (refs[0], refs[1])
    else:
        out_refs = refs
    b = pl.program_id(0)
    t = pl.program_id(1)
    g = G_ref[0]
    f = jax.nn.relu(g * a3_ref[...] + c3_ref[...])
    d = d_ref[0]
    e = jax.nn.relu(
        lax.dot_general(d, A2_ref[...], (((1,), (0,)), ((), ())),
                        preferred_element_type=jnp.float32) + c2_ref[...])
    m = e * f
    p = lax.dot_general(m, Wp_ref[...], (((1,), (0,)), ((), ())),
                        preferred_element_type=jnp.float32) + bp_ref[...]
    if not is_final:
        sp_ref, spp_ref = out_refs

        @pl.when(jnp.logical_and(b == 0, t == 0))
        def _():
            sp_ref[...] = jnp.zeros_like(sp_ref)
            spp_ref[...] = jnp.zeros_like(spp_ref)

        sp_ref[...] += p.sum(axis=0).reshape(1, GR)
        spp_ref[...] += (p * p).sum(axis=0).reshape(1, GR)
    else:
        x_ref, out_ref = out_refs
        r = jax.nn.relu(p)
        TN2 = r.shape[0] // K
        nf = r.reshape(TN2, K, GR).sum(axis=1)
        out_ref[0] = jnp.concatenate(
            [x_ref[0], nf.T], axis=0)


def _d_pass(is_final, G, d_all, feats, a3, c3, A2p, c2, W_post, bp, a4, c4):
    TN2 = 256
    TNK = TN2 * K
    nb = G.shape[0]
    in_specs = [
        pl.BlockSpec((1, TNK, 2 * GR), lambda b, t: (b, t, 0)),
        pl.BlockSpec((1, TNK, 4), lambda b, t: (b, t, 0)),
    ] + [pl.BlockSpec((1, 2 * GR), lambda b, t: (0, 0))] * 2       + [pl.BlockSpec((4, 2 * GR), lambda b, t: (0, 0)),
         pl.BlockSpec((1, 2 * GR), lambda b, t: (0, 0)),
         pl.BlockSpec((2 * GR, GR), lambda b, t: (0, 0))]       + [pl.BlockSpec((1, GR), lambda b, t: (0, 0))] * 3
    if is_final:
        in_specs.append(pl.BlockSpec((1, C_IN, TN2), lambda b, t: (b, 0, t)))
        out_specs = [pl.BlockSpec((1, C_IN + GR, TN2), lambda b, t: (b, 0, t))]
        out_shape = [jax.ShapeDtypeStruct((nb, C_IN + GR, N), jnp.float32)]
        args = (G, d_all, a3, c3, A2p, c2, W_post, bp, a4, c4, feats)
    else:
        out_specs = [pl.BlockSpec((1, GR), lambda b, t: (0, 0))] * 2
        out_shape = [jax.ShapeDtypeStruct((1, GR), jnp.float32)] * 2
        args = (G, d_all, a3, c3, A2p, c2, W_post, bp, a4, c4)
    res = pl.pallas_call(
        functools.partial(_d_body, is_final),
        grid=(nb, N // TN2),
        in_specs=in_specs,
        out_specs=out_specs,
        out_shape=out_shape,
    )(*args)
    return res


NW = 32
WPB = NW // B
RPW = NK // WPB
SC1_CH = 2048
SC2_CH = 256


def _wid():
    return lax.axis_index("s") * 2 + lax.axis_index("c")


def _sc1_body(idx_hbm, ptsp_hbm, cnt_hbm, d_hbm,
              idx_v, rows_v, pc_v, d_v, cnt_v, sem):
    w = _wid()
    b = w // WPB
    quarter = lax.rem(w, WPB)
    base = b * NK + quarter * RPW
    n0 = quarter * (N // WPB)
    ones = jnp.full((16,), 1.0, jnp.float32)
    zeros = jnp.zeros((16,), jnp.float32)
    iota = lax.iota(jnp.int32, 16)
    rowoff = iota // 4
    coloff = iota & 3

    def _zero(i, _):
        cnt_v[pl.ds(i * 16, 16)] = zeros
        return _
    lax.fori_loop(0, N // 16, _zero, None)

    def _chunk(ch, _):
        cbase = base + ch * SC1_CH
        pltpu.sync_copy(idx_hbm.at[pl.ds(cbase, SC1_CH)], idx_v)
        gat = pltpu.async_copy(ptsp_hbm.at[b].at[idx_v], rows_v, sem)

        def _hist(j, _):
            iv = idx_v[pl.ds(j * 16, 16)]
            plsc.addupdate_scatter(cnt_v, [iv], ones)
            return _
        lax.fori_loop(0, SC1_CH // 16, _hist, None)
        gat.wait()
        pltpu.sync_copy(
            ptsp_hbm.at[b].at[pl.ds(n0 + ch * (SC1_CH // K), SC1_CH // K)],
            pc_v)

        def _delta(n, _):
            pcv = plsc.load_gather(
                pc_v, [jnp.full((16,), n, jnp.int32), coloff])
            for j in range(4):
                rr = n * 16 + j * 4 + rowoff
                rv = plsc.load_gather(rows_v, [rr, coloff])
                plsc.store_scatter(d_v, [rr, coloff], rv - pcv)
            return _
        lax.fori_loop(0, SC1_CH // K, _delta, None)
        pltpu.sync_copy(d_v, d_hbm.at[b].at[pl.ds(quarter * RPW + ch * SC1_CH,
                                                  SC1_CH)])
        return _
    lax.fori_loop(0, RPW // SC1_CH, _chunk, None)
    pltpu.sync_copy(cnt_v, cnt_hbm.at[w])


def _sc1(knn_flat_1d, pts_pad16):
    mesh = plsc.VectorSubcoreMesh(core_axis_name="c", subcore_axis_name="s")
    return pl.kernel(
        _sc1_body,
        compiler_params=pltpu.CompilerParams(use_tc_tiling_on_sc=False, needs_layout_passes=False),
        out_type=[jax.ShapeDtypeStruct((NW, N), jnp.float32),
                  jax.ShapeDtypeStruct((B, NK, 4), jnp.float32)],
        mesh=mesh,
        scratch_types=[pltpu.VMEM((SC1_CH,), jnp.int32),
                       pltpu.VMEM((SC1_CH, 16), jnp.float32),
                       pltpu.VMEM((SC1_CH // K, 16), jnp.float32),
                       pltpu.VMEM((SC1_CH, 4), jnp.float32),
                       pltpu.VMEM((N,), jnp.float32),
                       pltpu.SemaphoreType.DMA],
    )(knn_flat_1d, pts_pad16)


def _sc2_body(nb, idx_hbm, T_hbm, G_hbm,
              idx_v0, idx_v1, rows_v0, rows_v1, sem0, sem1):
    w = _wid()
    wpb = NW // nb
    rpw = NK // wpb
    b = w // wpb
    part = lax.rem(w, wpb)
    base = b * NK + part * rpw
    lb = part * rpw
    nch = rpw // SC2_CH

    def _load(ch, idx_v, rows_v, sem):
        pltpu.sync_copy(idx_hbm.at[pl.ds(base + ch * SC2_CH, SC2_CH)], idx_v)
        pltpu.async_copy(T_hbm.at[b].at[idx_v], rows_v, sem)

    def _drain(idx_v, rows_v, sem):
        pltpu.make_async_copy(T_hbm.at[b].at[idx_v], rows_v, sem).wait()

    def _wb(ch, rows_v):
        pltpu.sync_copy(rows_v, G_hbm.at[b].at[pl.ds(lb + ch * SC2_CH,
                                                     SC2_CH)])

    _load(0, idx_v0, rows_v0, sem0)

    def _pair(j, _):
        _load(2 * j + 1, idx_v1, rows_v1, sem1)
        _drain(idx_v0, rows_v0, sem0)
        _wb(2 * j, rows_v0)

        @pl.when(j + 1 < nch // 2)
        def _():
            _load(2 * j + 2, idx_v0, rows_v0, sem0)
        _drain(idx_v1, rows_v1, sem1)
        _wb(2 * j + 1, rows_v1)
        return _
    lax.fori_loop(0, nch // 2, _pair, None)


def _sc2(knn_flat_1d, T):
    nb = T.shape[0]
    mesh = plsc.VectorSubcoreMesh(core_axis_name="c", subcore_axis_name="s")
    return pl.kernel(
        functools.partial(_sc2_body, nb),
        compiler_params=pltpu.CompilerParams(use_tc_tiling_on_sc=True, needs_layout_passes=False),
        out_type=jax.ShapeDtypeStruct((nb, NK, 2 * GR), jnp.float32),
        mesh=mesh,
        scratch_types=[pltpu.VMEM((SC2_CH,), jnp.int32),
                       pltpu.VMEM((SC2_CH,), jnp.int32),
                       pltpu.VMEM((SC2_CH, 2 * GR), jnp.float32),
                       pltpu.VMEM((SC2_CH, 2 * GR), jnp.float32),
                       pltpu.SemaphoreType.DMA,
                       pltpu.SemaphoreType.DMA],
    )(knn_flat_1d, T)


def kernel(feats, pts, knn_idx, W_bottle, b_bottle, g1, be1, W_delta, b_delta,
           g2, be2, W_feats, b_feats, g3, be3, W_post, b_post, g4, be4):
    knn_flat = knn_idx.reshape(B, NK).astype(jnp.int32)

    s1o, S1 = _feat_moments(feats)
    s1 = s1o[0]
    Ws = W_bottle @ s1
    sum_u = Ws + M1 * b_bottle
    sum_u2 = (jnp.einsum('oc,cd,od->o', W_bottle, S1, W_bottle)
              + 2 * b_bottle * Ws + M1 * b_bottle**2)
    mean1 = sum_u / M1
    var1 = sum_u2 / M1 - mean1**2
    a1 = g1 / jnp.sqrt(var1 + EPS)
    W1f = a1[:, None] * W_bottle
    c1f = (a1 * (b_bottle - mean1) + be1).reshape(1, C_IN)

    knn_1d = knn_flat.reshape(B * NK)
    pts_pad16 = jnp.concatenate(
        [pts.transpose(0, 2, 1), jnp.zeros((B, N, 13), jnp.float32)], axis=-1)
    counts32, d_all = _sc1(knn_1d, pts_pad16)
    counts4 = counts32.reshape(B, WPB, N)

    T, S3Q3, D1o, D2o = _table_and_stats(
        feats, counts4, d_all, W1f, c1f, W_feats, b_feats.reshape(1, GR))
    S3, Q3 = S3Q3[0], S3Q3[1]
    mean3 = S3 / M2
    var3 = Q3 / M2 - mean3**2
    a3 = (g3 / jnp.sqrt(var3 + EPS)).reshape(1, GR)
    c3 = (be3 - a3[0] * mean3).reshape(1, GR)

    D1 = D1o[0, :3]
    D2 = D2o[:3, :3]
    Wd1 = W_delta @ D1
    sum_z = Wd1 + M2 * b_delta
    sum_z2 = (jnp.einsum('oc,cd,od->o', W_delta, D2, W_delta)
              + 2 * b_delta * Wd1 + M2 * b_delta**2)
    mean2 = sum_z / M2
    var2 = sum_z2 / M2 - mean2**2
    a2 = g2 / jnp.sqrt(var2 + EPS)
    A2 = a2[:, None] * W_delta
    A2p = jnp.concatenate([A2.T, jnp.zeros((1, GR), jnp.float32)], axis=0)
    c2 = (a2 * (b_delta - mean2) + be2).reshape(1, GR)


    zz = jnp.zeros((1, GR), jnp.float32)
    z1 = jnp.zeros((1, GR), jnp.float32)
    a3p = jnp.concatenate([a3, z1], axis=1)
    c3p = jnp.concatenate([c3, z1], axis=1)
    A2pp = jnp.concatenate([A2p, jnp.zeros((4, GR), jnp.float32)], axis=1)
    c2p = jnp.concatenate([c2, z1], axis=1)
    Wp128 = jnp.concatenate([W_post.T, jnp.zeros((GR, GR), jnp.float32)],
                            axis=0)
    bp = b_post.reshape(1, GR)

    G = _sc2(knn_1d, T)
    sp, spp = _d_pass(False, G, d_all, None,
                      a3p, c3p, A2pp, c2p, Wp128, bp, zz, zz)
    mean4 = sp[0] / M2
    var4 = spp[0] / M2 - mean4**2
    a4 = (g4 / jnp.sqrt(var4 + EPS)).reshape(1, GR)
    c4 = (be4 - a4[0] * mean4).reshape(1, GR)

    Wp128f = Wp128 * a4
    bpf = (a4[0] * b_post + c4[0]).reshape(1, GR)
    (out,) = _d_pass(True, G, d_all, feats,
                     a3p, c3p, A2pp, c2p, Wp128f, bpf, a4, c4)
    return out


# speedup vs baseline: 1.0846x; 1.0846x over previous
import functools

import jax
import jax.numpy as jnp
from jax import lax
from jax.experimental import pallas as pl
from jax.experimental.pallas import tpu as pltpu
from jax.experimental.pallas import tpu_sc as plsc

B, C_IN, N, K, GR = 8, 128, 4096, 16, 64
NK = N * K
M1 = B * N
M2 = B * N * K
EPS = 1e-5


def _a_body(x_ref, s1_ref, S1_ref):
    b = pl.program_id(0)
    t = pl.program_id(1)
    x = x_ref[0]

    @pl.when(jnp.logical_and(b == 0, t == 0))
    def _():
        s1_ref[...] = jnp.zeros_like(s1_ref)
        S1_ref[...] = jnp.zeros_like(S1_ref)

    s1_ref[...] += x.sum(axis=1).reshape(1, C_IN)
    S1_ref[...] += lax.dot_general(x, x, (((1,), (1,)), ((), ())),
                                   preferred_element_type=jnp.float32)


def _feat_moments(feats):
    TN = 512
    return pl.pallas_call(
        _a_body,
        grid=(B, N // TN),
        in_specs=[pl.BlockSpec((1, C_IN, TN), lambda b, t: (b, 0, t))],
        out_specs=[pl.BlockSpec((1, C_IN), lambda b, t: (0, 0)),
                   pl.BlockSpec((C_IN, C_IN), lambda b, t: (0, 0))],
        out_shape=[jax.ShapeDtypeStruct((1, C_IN), jnp.float32),
                   jax.ShapeDtypeStruct((C_IN, C_IN), jnp.float32)],
    )(feats)


def _c_body(x_ref, cnt_ref, d_ref, W1f_ref, c1f_ref, Wf_ref, bf_ref,
            T_ref, S3Q3_ref, D1_ref, D2_ref):
    b = pl.program_id(0)
    t = pl.program_id(1)

    @pl.when(jnp.logical_and(b == 0, t == 0))
    def _():
        S3Q3_ref[...] = jnp.zeros_like(S3Q3_ref)
        D1_ref[...] = jnp.zeros_like(D1_ref)
        D2_ref[...] = jnp.zeros_like(D2_ref)

    x = x_ref[0]
    nf = jax.nn.relu(
        lax.dot_general(x, W1f_ref[...], (((0,), (1,)), ((), ())),
                        preferred_element_type=jnp.float32)
        + c1f_ref[...])
    h = lax.dot_general(nf, Wf_ref[...], (((1,), (1,)), ((), ())),
                        preferred_element_type=jnp.float32) + bf_ref[...]
    T_ref[0] = jnp.concatenate(
        [h, jnp.zeros_like(h)], axis=1)

    cnt = cnt_ref[0].sum(axis=0).reshape(-1, 1)
    hw = h * cnt
    S3 = hw.sum(axis=0).reshape(1, GR)
    Q3 = (hw * h).sum(axis=0).reshape(1, GR)
    S3Q3_ref[...] += jnp.concatenate([S3, Q3], axis=0)

    d = d_ref[0]
    D1_ref[...] += d.sum(axis=0).reshape(1, 4)
    D2_ref[...] += lax.dot_general(d, d, (((0,), (0,)), ((), ())),
                                   preferred_element_type=jnp.float32)


def _table_and_stats(feats, counts4, d_all, W1f, c1f, W_feats, b_feats):
    TN = 512
    return pl.pallas_call(
        _c_body,
        grid=(B, N // TN),
        in_specs=[
            pl.BlockSpec((1, C_IN, TN), lambda b, t: (b, 0, t)),
            pl.BlockSpec((1, 4, TN), lambda b, t: (b, 0, t)),
            pl.BlockSpec((1, TN * K, 4), lambda b, t: (b, t, 0)),
            pl.BlockSpec((C_IN, C_IN), lambda b, t: (0, 0)),
            pl.BlockSpec((1, C_IN), lambda b, t: (0, 0)),
            pl.BlockSpec((GR, C_IN), lambda b, t: (0, 0)),
            pl.BlockSpec((1, GR), lambda b, t: (0, 0)),
        ],
        out_specs=[
            pl.BlockSpec((1, TN, 2 * GR), lambda b, t: (b, t, 0)),
            pl.BlockSpec((2, GR), lambda b, t: (0, 0)),
            pl.BlockSpec((1, 4), lambda b, t: (0, 0)),
            pl.BlockSpec((4, 4), lambda b, t: (0, 0)),
        ],
        out_shape=[
            jax.ShapeDtypeStruct((B, N, 2 * GR), jnp.float32),
            jax.ShapeDtypeStruct((2, GR), jnp.float32),
            jax.ShapeDtypeStruct((1, 4), jnp.float32),
            jax.ShapeDtypeStruct((4, 4), jnp.float32),
        ],
    )(feats, counts4, d_all, W1f, c1f, W_feats, b_feats)


def _d_body(is_final, G_ref, d_ref, a3_ref, c3_ref, A2_ref, c2_ref,
            Wp_ref, bp_ref, a4_ref, c4_ref, *refs):
    if is_final:
        out_refs = (refs[0], refs[1])
    else:
        out_refs = refs
    b = pl.program_id(0)
    t = pl.program_id(1)
    g = G_ref[0]
    f = jax.nn.relu(g * a3_ref[...] + c3_ref[...])
    d = d_ref[0]
    e = jax.nn.relu(
        lax.dot_general(d, A2_ref[...], (((1,), (0,)), ((), ())),
                        preferred_element_type=jnp.float32) + c2_ref[...])
    m = e * f
    p = lax.dot_general(m, Wp_ref[...], (((1,), (0,)), ((), ())),
                        preferred_element_type=jnp.float32) + bp_ref[...]
    if not is_final:
        sp_ref, spp_ref = out_refs

        @pl.when(jnp.logical_and(b == 0, t == 0))
        def _():
            sp_ref[...] = jnp.zeros_like(sp_ref)
            spp_ref[...] = jnp.zeros_like(spp_ref)

        sp_ref[...] += p.sum(axis=0).reshape(1, GR)
        spp_ref[...] += (p * p).sum(axis=0).reshape(1, GR)
    else:
        x_ref, out_ref = out_refs
        r = jax.nn.relu(p)
        TN2 = r.shape[0] // K
        nf = r.reshape(TN2, K, GR).sum(axis=1)
        out_ref[0] = jnp.concatenate(
            [x_ref[0], nf.T], axis=0)


def _d_pass(is_final, G, d_all, feats, a3, c3, A2p, c2, W_post, bp, a4, c4):
    TN2 = 512
    TNK = TN2 * K
    nb = G.shape[0]
    in_specs = [
        pl.BlockSpec((1, TNK, 2 * GR), lambda b, t: (b, t, 0)),
        pl.BlockSpec((1, TNK, 4), lambda b, t: (b, t, 0)),
    ] + [pl.BlockSpec((1, 2 * GR), lambda b, t: (0, 0))] * 2       + [pl.BlockSpec((4, 2 * GR), lambda b, t: (0, 0)),
         pl.BlockSpec((1, 2 * GR), lambda b, t: (0, 0)),
         pl.BlockSpec((2 * GR, GR), lambda b, t: (0, 0))]       + [pl.BlockSpec((1, GR), lambda b, t: (0, 0))] * 3
    if is_final:
        in_specs.append(pl.BlockSpec((1, C_IN, TN2), lambda b, t: (b, 0, t)))
        out_specs = [pl.BlockSpec((1, C_IN + GR, TN2), lambda b, t: (b, 0, t))]
        out_shape = [jax.ShapeDtypeStruct((nb, C_IN + GR, N), jnp.float32)]
        args = (G, d_all, a3, c3, A2p, c2, W_post, bp, a4, c4, feats)
    else:
        out_specs = [pl.BlockSpec((1, GR), lambda b, t: (0, 0))] * 2
        out_shape = [jax.ShapeDtypeStruct((1, GR), jnp.float32)] * 2
        args = (G, d_all, a3, c3, A2p, c2, W_post, bp, a4, c4)
    res = pl.pallas_call(
        functools.partial(_d_body, is_final),
        grid=(nb, N // TN2),
        in_specs=in_specs,
        out_specs=out_specs,
        out_shape=out_shape,
    )(*args)
    return res


NW = 32
WPB = NW // B
RPW = NK // WPB
SC1_CH = 2048
SC2_CH = 256


def _wid():
    return lax.axis_index("s") * 2 + lax.axis_index("c")


def _sc1_body(idx_hbm, ptsp_hbm, cnt_hbm, d_hbm,
              idx_v, rows_v, pc_v, d_v, cnt_v, sem):
    w = _wid()
    b = w // WPB
    quarter = lax.rem(w, WPB)
    base = b * NK + quarter * RPW
    n0 = quarter * (N // WPB)
    ones = jnp.full((16,), 1.0, jnp.float32)
    zeros = jnp.zeros((16,), jnp.float32)
    iota = lax.iota(jnp.int32, 16)
    rowoff = iota // 4
    coloff = iota & 3

    def _zero(i, _):
        cnt_v[pl.ds(i * 16, 16)] = zeros
        return _
    lax.fori_loop(0, N // 16, _zero, None)

    def _chunk(ch, _):
        cbase = base + ch * SC1_CH
        pltpu.sync_copy(idx_hbm.at[pl.ds(cbase, SC1_CH)], idx_v)
        gat = pltpu.async_copy(ptsp_hbm.at[b].at[idx_v], rows_v, sem)

        def _hist(j, _):
            iv = idx_v[pl.ds(j * 16, 16)]
            plsc.addupdate_scatter(cnt_v, [iv], ones)
            return _
        lax.fori_loop(0, SC1_CH // 16, _hist, None)
        gat.wait()
        pltpu.sync_copy(
            ptsp_hbm.at[b].at[pl.ds(n0 + ch * (SC1_CH // K), SC1_CH // K)],
            pc_v)

        def _delta(n, _):
            pcv = plsc.load_gather(
                pc_v, [jnp.full((16,), n, jnp.int32), coloff])
            for j in range(4):
                rr = n * 16 + j * 4 + rowoff
                rv = plsc.load_gather(rows_v, [rr, coloff])
                plsc.store_scatter(d_v, [rr, coloff], rv - pcv)
            return _
        lax.fori_loop(0, SC1_CH // K, _delta, None)
        pltpu.sync_copy(d_v, d_hbm.at[b].at[pl.ds(quarter * RPW + ch * SC1_CH,
                                                  SC1_CH)])
        return _
    lax.fori_loop(0, RPW // SC1_CH, _chunk, None)
    pltpu.sync_copy(cnt_v, cnt_hbm.at[w])


def _sc1(knn_flat_1d, pts_pad16):
    mesh = plsc.VectorSubcoreMesh(core_axis_name="c", subcore_axis_name="s")
    return pl.kernel(
        _sc1_body,
        compiler_params=pltpu.CompilerParams(use_tc_tiling_on_sc=False, needs_layout_passes=False),
        out_type=[jax.ShapeDtypeStruct((NW, N), jnp.float32),
                  jax.ShapeDtypeStruct((B, NK, 4), jnp.float32)],
        mesh=mesh,
        scratch_types=[pltpu.VMEM((SC1_CH,), jnp.int32),
                       pltpu.VMEM((SC1_CH, 16), jnp.float32),
                       pltpu.VMEM((SC1_CH // K, 16), jnp.float32),
                       pltpu.VMEM((SC1_CH, 4), jnp.float32),
                       pltpu.VMEM((N,), jnp.float32),
                       pltpu.SemaphoreType.DMA],
    )(knn_flat_1d, pts_pad16)


def _sc2_body(nb, idx_hbm, T_hbm, G_hbm,
              idx_v0, idx_v1, rows_v0, rows_v1, sem0, sem1):
    w = _wid()
    wpb = NW // nb
    rpw = NK // wpb
    b = w // wpb
    part = lax.rem(w, wpb)
    base = b * NK + part * rpw
    lb = part * rpw
    nch = rpw // SC2_CH

    def _load(ch, idx_v, rows_v, sem):
        pltpu.sync_copy(idx_hbm.at[pl.ds(base + ch * SC2_CH, SC2_CH)], idx_v)
        pltpu.async_copy(T_hbm.at[b].at[idx_v], rows_v, sem)

    def _drain(idx_v, rows_v, sem):
        pltpu.make_async_copy(T_hbm.at[b].at[idx_v], rows_v, sem).wait()

    def _wb(ch, rows_v):
        pltpu.sync_copy(rows_v, G_hbm.at[b].at[pl.ds(lb + ch * SC2_CH,
                                                     SC2_CH)])

    _load(0, idx_v0, rows_v0, sem0)

    def _pair(j, _):
        _load(2 * j + 1, idx_v1, rows_v1, sem1)
        _drain(idx_v0, rows_v0, sem0)
        _wb(2 * j, rows_v0)

        @pl.when(j + 1 < nch // 2)
        def _():
            _load(2 * j + 2, idx_v0, rows_v0, sem0)
        _drain(idx_v1, rows_v1, sem1)
        _wb(2 * j + 1, rows_v1)
        return _
    lax.fori_loop(0, nch // 2, _pair, None)


def _sc2(knn_flat_1d, T):
    nb = T.shape[0]
    mesh = plsc.VectorSubcoreMesh(core_axis_name="c", subcore_axis_name="s")
    return pl.kernel(
        functools.partial(_sc2_body, nb),
        compiler_params=pltpu.CompilerParams(use_tc_tiling_on_sc=True, needs_layout_passes=False),
        out_type=jax.ShapeDtypeStruct((nb, NK, 2 * GR), jnp.float32),
        mesh=mesh,
        scratch_types=[pltpu.VMEM((SC2_CH,), jnp.int32),
                       pltpu.VMEM((SC2_CH,), jnp.int32),
                       pltpu.VMEM((SC2_CH, 2 * GR), jnp.float32),
                       pltpu.VMEM((SC2_CH, 2 * GR), jnp.float32),
                       pltpu.SemaphoreType.DMA,
                       pltpu.SemaphoreType.DMA],
    )(knn_flat_1d, T)


def kernel(feats, pts, knn_idx, W_bottle, b_bottle, g1, be1, W_delta, b_delta,
           g2, be2, W_feats, b_feats, g3, be3, W_post, b_post, g4, be4):
    knn_flat = knn_idx.reshape(B, NK).astype(jnp.int32)

    s1o, S1 = _feat_moments(feats)
    s1 = s1o[0]
    Ws = W_bottle @ s1
    sum_u = Ws + M1 * b_bottle
    sum_u2 = (jnp.einsum('oc,cd,od->o', W_bottle, S1, W_bottle)
              + 2 * b_bottle * Ws + M1 * b_bottle**2)
    mean1 = sum_u / M1
    var1 = sum_u2 / M1 - mean1**2
    a1 = g1 / jnp.sqrt(var1 + EPS)
    W1f = a1[:, None] * W_bottle
    c1f = (a1 * (b_bottle - mean1) + be1).reshape(1, C_IN)

    knn_1d = knn_flat.reshape(B * NK)
    pts_pad16 = jnp.concatenate(
        [pts.transpose(0, 2, 1), jnp.zeros((B, N, 13), jnp.float32)], axis=-1)
    counts32, d_all = _sc1(knn_1d, pts_pad16)
    counts4 = counts32.reshape(B, WPB, N)

    T, S3Q3, D1o, D2o = _table_and_stats(
        feats, counts4, d_all, W1f, c1f, W_feats, b_feats.reshape(1, GR))
    S3, Q3 = S3Q3[0], S3Q3[1]
    mean3 = S3 / M2
    var3 = Q3 / M2 - mean3**2
    a3 = (g3 / jnp.sqrt(var3 + EPS)).reshape(1, GR)
    c3 = (be3 - a3[0] * mean3).reshape(1, GR)

    D1 = D1o[0, :3]
    D2 = D2o[:3, :3]
    Wd1 = W_delta @ D1
    sum_z = Wd1 + M2 * b_delta
    sum_z2 = (jnp.einsum('oc,cd,od->o', W_delta, D2, W_delta)
              + 2 * b_delta * Wd1 + M2 * b_delta**2)
    mean2 = sum_z / M2
    var2 = sum_z2 / M2 - mean2**2
    a2 = g2 / jnp.sqrt(var2 + EPS)
    A2 = a2[:, None] * W_delta
    A2p = jnp.concatenate([A2.T, jnp.zeros((1, GR), jnp.float32)], axis=0)
    c2 = (a2 * (b_delta - mean2) + be2).reshape(1, GR)


    zz = jnp.zeros((1, GR), jnp.float32)
    z1 = jnp.zeros((1, GR), jnp.float32)
    a3p = jnp.concatenate([a3, z1], axis=1)
    c3p = jnp.concatenate([c3, z1], axis=1)
    A2pp = jnp.concatenate([A2p, jnp.zeros((4, GR), jnp.float32)], axis=1)
    c2p = jnp.concatenate([c2, z1], axis=1)
    Wp128 = jnp.concatenate([W_post.T, jnp.zeros((GR, GR), jnp.float32)],
                            axis=0)
    bp = b_post.reshape(1, GR)

    G = _sc2(knn_1d, T)
    sp, spp = _d_pass(False, G, d_all, None,
                      a3p, c3p, A2pp, c2p, Wp128, bp, zz, zz)
    mean4 = sp[0] / M2
    var4 = spp[0] / M2 - mean4**2
    a4 = (g4 / jnp.sqrt(var4 + EPS)).reshape(1, GR)
    c4 = (be4 - a4[0] * mean4).reshape(1, GR)

    Wp128f = Wp128 * a4
    bpf = (a4[0] * b_post + c4[0]).reshape(1, GR)
    (out,) = _d_pass(True, G, d_all, feats,
                     a3p, c3p, A2pp, c2p, Wp128f, bpf, a4, c4)
    return out


# speedup vs baseline: 1.1193x; 1.0319x over previous
import functools

import jax
import jax.numpy as jnp
from jax import lax
from jax.experimental import pallas as pl
from jax.experimental.pallas import tpu as pltpu
from jax.experimental.pallas import tpu_sc as plsc

B, C_IN, N, K, GR = 8, 128, 4096, 16, 64
NK = N * K
M1 = B * N
M2 = B * N * K
EPS = 1e-5


def _a_body(x_ref, s1_ref, S1_ref):
    b = pl.program_id(0)
    t = pl.program_id(1)
    x = x_ref[0]

    @pl.when(jnp.logical_and(b == 0, t == 0))
    def _():
        s1_ref[...] = jnp.zeros_like(s1_ref)
        S1_ref[...] = jnp.zeros_like(S1_ref)

    s1_ref[...] += x.sum(axis=1).reshape(1, C_IN)
    S1_ref[...] += lax.dot_general(x, x, (((1,), (1,)), ((), ())),
                                   preferred_element_type=jnp.float32)


def _feat_moments(feats):
    TN = 512
    return pl.pallas_call(
        _a_body,
        grid=(B, N // TN),
        in_specs=[pl.BlockSpec((1, C_IN, TN), lambda b, t: (b, 0, t))],
        out_specs=[pl.BlockSpec((1, C_IN), lambda b, t: (0, 0)),
                   pl.BlockSpec((C_IN, C_IN), lambda b, t: (0, 0))],
        out_shape=[jax.ShapeDtypeStruct((1, C_IN), jnp.float32),
                   jax.ShapeDtypeStruct((C_IN, C_IN), jnp.float32)],
    )(feats)


def _c_body(x_ref, cnt_ref, d_ref, W1f_ref, c1f_ref, Wf_ref, bf_ref,
            T_ref, S3Q3_ref, D1_ref, D2_ref):
    b = pl.program_id(0)
    t = pl.program_id(1)

    @pl.when(jnp.logical_and(b == 0, t == 0))
    def _():
        S3Q3_ref[...] = jnp.zeros_like(S3Q3_ref)
        D1_ref[...] = jnp.zeros_like(D1_ref)
        D2_ref[...] = jnp.zeros_like(D2_ref)

    x = x_ref[0]
    nf = jax.nn.relu(
        lax.dot_general(x, W1f_ref[...], (((0,), (1,)), ((), ())),
                        preferred_element_type=jnp.float32)
        + c1f_ref[...])
    h = lax.dot_general(nf, Wf_ref[...], (((1,), (1,)), ((), ())),
                        preferred_element_type=jnp.float32) + bf_ref[...]
    T_ref[0] = jnp.concatenate(
        [h, jnp.zeros_like(h)], axis=1)

    cnt = cnt_ref[0].sum(axis=0).reshape(-1, 1)
    hw = h * cnt
    S3 = hw.sum(axis=0).reshape(1, GR)
    Q3 = (hw * h).sum(axis=0).reshape(1, GR)
    S3Q3_ref[...] += jnp.concatenate([S3, Q3], axis=0)

    d = d_ref[0]
    D1_ref[...] += d.sum(axis=0).reshape(1, 4)
    D2_ref[...] += lax.dot_general(d, d, (((0,), (0,)), ((), ())),
                                   preferred_element_type=jnp.float32)


def _table_and_stats(feats, counts4, d_all, W1f, c1f, W_feats, b_feats):
    TN = 512
    return pl.pallas_call(
        _c_body,
        grid=(B, N // TN),
        in_specs=[
            pl.BlockSpec((1, C_IN, TN), lambda b, t: (b, 0, t)),
            pl.BlockSpec((1, 4, TN), lambda b, t: (b, 0, t)),
            pl.BlockSpec((1, TN * K, 4), lambda b, t: (b, t, 0)),
            pl.BlockSpec((C_IN, C_IN), lambda b, t: (0, 0)),
            pl.BlockSpec((1, C_IN), lambda b, t: (0, 0)),
            pl.BlockSpec((GR, C_IN), lambda b, t: (0, 0)),
            pl.BlockSpec((1, GR), lambda b, t: (0, 0)),
        ],
        out_specs=[
            pl.BlockSpec((1, TN, 2 * GR), lambda b, t: (b, t, 0)),
            pl.BlockSpec((2, GR), lambda b, t: (0, 0)),
            pl.BlockSpec((1, 4), lambda b, t: (0, 0)),
            pl.BlockSpec((4, 4), lambda b, t: (0, 0)),
        ],
        out_shape=[
            jax.ShapeDtypeStruct((B, N, 2 * GR), jnp.float32),
            jax.ShapeDtypeStruct((2, GR), jnp.float32),
            jax.ShapeDtypeStruct((1, 4), jnp.float32),
            jax.ShapeDtypeStruct((4, 4), jnp.float32),
        ],
    )(feats, counts4, d_all, W1f, c1f, W_feats, b_feats)


def _d_body(is_final, G_ref, d_ref, a3_ref, c3_ref, A2_ref, c2_ref,
            Wp_ref, bp_ref, a4_ref, c4_ref, *refs):
    if is_final:
        out_refs = (refs[0], refs[1])
    else:
        out_refs = refs
    b = pl.program_id(0)
    t = pl.program_id(1)
    g = G_ref[0]
    f = jax.nn.relu(g * a3_ref[...] + c3_ref[...])
    d = d_ref[0]
    e = jax.nn.relu(
        lax.dot_general(d, A2_ref[...], (((1,), (0,)), ((), ())),
                        preferred_element_type=jnp.float32) + c2_ref[...])
    m = e * f
    p = lax.dot_general(m, Wp_ref[...], (((1,), (0,)), ((), ())),
                        preferred_element_type=jnp.float32) + bp_ref[...]
    if not is_final:
        sp_ref, spp_ref = out_refs

        @pl.when(jnp.logical_and(b == 0, t == 0))
        def _():
            sp_ref[...] = jnp.zeros_like(sp_ref)
            spp_ref[...] = jnp.zeros_like(spp_ref)

        sp_ref[...] += p.sum(axis=0).reshape(1, GR)
        spp_ref[...] += (p * p).sum(axis=0).reshape(1, GR)
    else:
        x_ref, out_ref = out_refs
        r = jax.nn.relu(p)
        TN2 = r.shape[0] // K
        nf = r.reshape(TN2, K, GR).sum(axis=1)
        out_ref[0] = jnp.concatenate(
            [x_ref[0], nf.T], axis=0)


def _d_pass(is_final, G, d_all, feats, a3, c3, A2p, c2, W_post, bp, a4, c4):
    TN2 = 1024
    TNK = TN2 * K
    nb = G.shape[0]
    in_specs = [
        pl.BlockSpec((1, TNK, 2 * GR), lambda b, t: (b, t, 0)),
        pl.BlockSpec((1, TNK, 4), lambda b, t: (b, t, 0)),
    ] + [pl.BlockSpec((1, 2 * GR), lambda b, t: (0, 0))] * 2       + [pl.BlockSpec((4, 2 * GR), lambda b, t: (0, 0)),
         pl.BlockSpec((1, 2 * GR), lambda b, t: (0, 0)),
         pl.BlockSpec((2 * GR, GR), lambda b, t: (0, 0))]       + [pl.BlockSpec((1, GR), lambda b, t: (0, 0))] * 3
    if is_final:
        in_specs.append(pl.BlockSpec((1, C_IN, TN2), lambda b, t: (b, 0, t)))
        out_specs = [pl.BlockSpec((1, C_IN + GR, TN2), lambda b, t: (b, 0, t))]
        out_shape = [jax.ShapeDtypeStruct((nb, C_IN + GR, N), jnp.float32)]
        args = (G, d_all, a3, c3, A2p, c2, W_post, bp, a4, c4, feats)
    else:
        out_specs = [pl.BlockSpec((1, GR), lambda b, t: (0, 0))] * 2
        out_shape = [jax.ShapeDtypeStruct((1, GR), jnp.float32)] * 2
        args = (G, d_all, a3, c3, A2p, c2, W_post, bp, a4, c4)
    res = pl.pallas_call(
        functools.partial(_d_body, is_final),
        grid=(nb, N // TN2),
        in_specs=in_specs,
        out_specs=out_specs,
        out_shape=out_shape,
    )(*args)
    return res


NW = 32
WPB = NW // B
RPW = NK // WPB
SC1_CH = 2048
SC2_CH = 256


def _wid():
    return lax.axis_index("s") * 2 + lax.axis_index("c")


def _sc1_body(idx_hbm, ptsp_hbm, cnt_hbm, d_hbm,
              idx_v, rows_v, pc_v, d_v, cnt_v, sem):
    w = _wid()
    b = w // WPB
    quarter = lax.rem(w, WPB)
    base = b * NK + quarter * RPW
    n0 = quarter * (N // WPB)
    ones = jnp.full((16,), 1.0, jnp.float32)
    zeros = jnp.zeros((16,), jnp.float32)
    iota = lax.iota(jnp.int32, 16)
    rowoff = iota // 4
    coloff = iota & 3

    def _zero(i, _):
        cnt_v[pl.ds(i * 16, 16)] = zeros
        return _
    lax.fori_loop(0, N // 16, _zero, None)

    def _chunk(ch, _):
        cbase = base + ch * SC1_CH
        pltpu.sync_copy(idx_hbm.at[pl.ds(cbase, SC1_CH)], idx_v)
        gat = pltpu.async_copy(ptsp_hbm.at[b].at[idx_v], rows_v, sem)

        def _hist(j, _):
            iv = idx_v[pl.ds(j * 16, 16)]
            plsc.addupdate_scatter(cnt_v, [iv], ones)
            return _
        lax.fori_loop(0, SC1_CH // 16, _hist, None)
        gat.wait()
        pltpu.sync_copy(
            ptsp_hbm.at[b].at[pl.ds(n0 + ch * (SC1_CH // K), SC1_CH // K)],
            pc_v)

        def _delta(n, _):
            pcv = plsc.load_gather(
                pc_v, [jnp.full((16,), n, jnp.int32), coloff])
            for j in range(4):
                rr = n * 16 + j * 4 + rowoff
                rv = plsc.load_gather(rows_v, [rr, coloff])
                plsc.store_scatter(d_v, [rr, coloff], rv - pcv)
            return _
        lax.fori_loop(0, SC1_CH // K, _delta, None)
        pltpu.sync_copy(d_v, d_hbm.at[b].at[pl.ds(quarter * RPW + ch * SC1_CH,
                                                  SC1_CH)])
        return _
    lax.fori_loop(0, RPW // SC1_CH, _chunk, None)
    pltpu.sync_copy(cnt_v, cnt_hbm.at[w])


def _sc1(knn_flat_1d, pts_pad16):
    mesh = plsc.VectorSubcoreMesh(core_axis_name="c", subcore_axis_name="s")
    return pl.kernel(
        _sc1_body,
        compiler_params=pltpu.CompilerParams(use_tc_tiling_on_sc=False, needs_layout_passes=False),
        out_type=[jax.ShapeDtypeStruct((NW, N), jnp.float32),
                  jax.ShapeDtypeStruct((B, NK, 4), jnp.float32)],
        mesh=mesh,
        scratch_types=[pltpu.VMEM((SC1_CH,), jnp.int32),
                       pltpu.VMEM((SC1_CH, 16), jnp.float32),
                       pltpu.VMEM((SC1_CH // K, 16), jnp.float32),
                       pltpu.VMEM((SC1_CH, 4), jnp.float32),
                       pltpu.VMEM((N,), jnp.float32),
                       pltpu.SemaphoreType.DMA],
    )(knn_flat_1d, pts_pad16)


def _sc2_body(nb, idx_hbm, T_hbm, G_hbm,
              idx_v0, idx_v1, rows_v0, rows_v1, sem0, sem1):
    w = _wid()
    wpb = NW // nb
    rpw = NK // wpb
    b = w // wpb
    part = lax.rem(w, wpb)
    base = b * NK + part * rpw
    lb = part * rpw
    nch = rpw // SC2_CH

    def _load(ch, idx_v, rows_v, sem):
        pltpu.sync_copy(idx_hbm.at[pl.ds(base + ch * SC2_CH, SC2_CH)], idx_v)
        pltpu.async_copy(T_hbm.at[b].at[idx_v], rows_v, sem)

    def _drain(idx_v, rows_v, sem):
        pltpu.make_async_copy(T_hbm.at[b].at[idx_v], rows_v, sem).wait()

    def _wb(ch, rows_v):
        pltpu.sync_copy(rows_v, G_hbm.at[b].at[pl.ds(lb + ch * SC2_CH,
                                                     SC2_CH)])

    _load(0, idx_v0, rows_v0, sem0)

    def _pair(j, _):
        _load(2 * j + 1, idx_v1, rows_v1, sem1)
        _drain(idx_v0, rows_v0, sem0)
        _wb(2 * j, rows_v0)

        @pl.when(j + 1 < nch // 2)
        def _():
            _load(2 * j + 2, idx_v0, rows_v0, sem0)
        _drain(idx_v1, rows_v1, sem1)
        _wb(2 * j + 1, rows_v1)
        return _
    lax.fori_loop(0, nch // 2, _pair, None)


def _sc2(knn_flat_1d, T):
    nb = T.shape[0]
    mesh = plsc.VectorSubcoreMesh(core_axis_name="c", subcore_axis_name="s")
    return pl.kernel(
        functools.partial(_sc2_body, nb),
        compiler_params=pltpu.CompilerParams(use_tc_tiling_on_sc=True, needs_layout_passes=False),
        out_type=jax.ShapeDtypeStruct((nb, NK, 2 * GR), jnp.float32),
        mesh=mesh,
        scratch_types=[pltpu.VMEM((SC2_CH,), jnp.int32),
                       pltpu.VMEM((SC2_CH,), jnp.int32),
                       pltpu.VMEM((SC2_CH, 2 * GR), jnp.float32),
                       pltpu.VMEM((SC2_CH, 2 * GR), jnp.float32),
                       pltpu.SemaphoreType.DMA,
                       pltpu.SemaphoreType.DMA],
    )(knn_flat_1d, T)


def kernel(feats, pts, knn_idx, W_bottle, b_bottle, g1, be1, W_delta, b_delta,
           g2, be2, W_feats, b_feats, g3, be3, W_post, b_post, g4, be4):
    knn_flat = knn_idx.reshape(B, NK).astype(jnp.int32)

    s1o, S1 = _feat_moments(feats)
    s1 = s1o[0]
    Ws = W_bottle @ s1
    sum_u = Ws + M1 * b_bottle
    sum_u2 = (jnp.einsum('oc,cd,od->o', W_bottle, S1, W_bottle)
              + 2 * b_bottle * Ws + M1 * b_bottle**2)
    mean1 = sum_u / M1
    var1 = sum_u2 / M1 - mean1**2
    a1 = g1 / jnp.sqrt(var1 + EPS)
    W1f = a1[:, None] * W_bottle
    c1f = (a1 * (b_bottle - mean1) + be1).reshape(1, C_IN)

    knn_1d = knn_flat.reshape(B * NK)
    pts_pad16 = jnp.concatenate(
        [pts.transpose(0, 2, 1), jnp.zeros((B, N, 13), jnp.float32)], axis=-1)
    counts32, d_all = _sc1(knn_1d, pts_pad16)
    counts4 = counts32.reshape(B, WPB, N)

    T, S3Q3, D1o, D2o = _table_and_stats(
        feats, counts4, d_all, W1f, c1f, W_feats, b_feats.reshape(1, GR))
    S3, Q3 = S3Q3[0], S3Q3[1]
    mean3 = S3 / M2
    var3 = Q3 / M2 - mean3**2
    a3 = (g3 / jnp.sqrt(var3 + EPS)).reshape(1, GR)
    c3 = (be3 - a3[0] * mean3).reshape(1, GR)

    D1 = D1o[0, :3]
    D2 = D2o[:3, :3]
    Wd1 = W_delta @ D1
    sum_z = Wd1 + M2 * b_delta
    sum_z2 = (jnp.einsum('oc,cd,od->o', W_delta, D2, W_delta)
              + 2 * b_delta * Wd1 + M2 * b_delta**2)
    mean2 = sum_z / M2
    var2 = sum_z2 / M2 - mean2**2
    a2 = g2 / jnp.sqrt(var2 + EPS)
    A2 = a2[:, None] * W_delta
    A2p = jnp.concatenate([A2.T, jnp.zeros((1, GR), jnp.float32)], axis=0)
    c2 = (a2 * (b_delta - mean2) + be2).reshape(1, GR)


    zz = jnp.zeros((1, GR), jnp.float32)
    z1 = jnp.zeros((1, GR), jnp.float32)
    a3p = jnp.concatenate([a3, z1], axis=1)
    c3p = jnp.concatenate([c3, z1], axis=1)
    A2pp = jnp.concatenate([A2p, jnp.zeros((4, GR), jnp.float32)], axis=1)
    c2p = jnp.concatenate([c2, z1], axis=1)
    Wp128 = jnp.concatenate([W_post.T, jnp.zeros((GR, GR), jnp.float32)],
                            axis=0)
    bp = b_post.reshape(1, GR)

    G = _sc2(knn_1d, T)
    sp, spp = _d_pass(False, G, d_all, None,
                      a3p, c3p, A2pp, c2p, Wp128, bp, zz, zz)
    mean4 = sp[0] / M2
    var4 = spp[0] / M2 - mean4**2
    a4 = (g4 / jnp.sqrt(var4 + EPS)).reshape(1, GR)
    c4 = (be4 - a4[0] * mean4).reshape(1, GR)

    Wp128f = Wp128 * a4
    bpf = (a4[0] * b_post + c4[0]).reshape(1, GR)
    (out,) = _d_pass(True, G, d_all, feats,
                     a3p, c3p, A2pp, c2p, Wp128f, bpf, a4, c4)
    return out


# speedup vs baseline: 1.1365x; 1.0153x over previous
import functools

import jax
import jax.numpy as jnp
from jax import lax
from jax.experimental import pallas as pl
from jax.experimental.pallas import tpu as pltpu
from jax.experimental.pallas import tpu_sc as plsc

B, C_IN, N, K, GR = 8, 128, 4096, 16, 64
NK = N * K
M1 = B * N
M2 = B * N * K
EPS = 1e-5


def _a_body(x_ref, s1_ref, S1_ref):
    b = pl.program_id(0)
    t = pl.program_id(1)
    x = x_ref[0]

    @pl.when(jnp.logical_and(b == 0, t == 0))
    def _():
        s1_ref[...] = jnp.zeros_like(s1_ref)
        S1_ref[...] = jnp.zeros_like(S1_ref)

    s1_ref[...] += x.sum(axis=1).reshape(1, C_IN)
    S1_ref[...] += lax.dot_general(x, x, (((1,), (1,)), ((), ())),
                                   preferred_element_type=jnp.float32)


def _feat_moments(feats):
    TN = 512
    return pl.pallas_call(
        _a_body,
        grid=(B, N // TN),
        in_specs=[pl.BlockSpec((1, C_IN, TN), lambda b, t: (b, 0, t))],
        out_specs=[pl.BlockSpec((1, C_IN), lambda b, t: (0, 0)),
                   pl.BlockSpec((C_IN, C_IN), lambda b, t: (0, 0))],
        out_shape=[jax.ShapeDtypeStruct((1, C_IN), jnp.float32),
                   jax.ShapeDtypeStruct((C_IN, C_IN), jnp.float32)],
    )(feats)


def _c_body(x_ref, cnt_ref, d_ref, W1f_ref, c1f_ref, Wf_ref, bf_ref,
            T_ref, S3Q3_ref, D1_ref, D2_ref):
    b = pl.program_id(0)
    t = pl.program_id(1)

    @pl.when(jnp.logical_and(b == 0, t == 0))
    def _():
        S3Q3_ref[...] = jnp.zeros_like(S3Q3_ref)
        D1_ref[...] = jnp.zeros_like(D1_ref)
        D2_ref[...] = jnp.zeros_like(D2_ref)

    x = x_ref[0]
    nf = jax.nn.relu(
        lax.dot_general(x, W1f_ref[...], (((0,), (1,)), ((), ())),
                        preferred_element_type=jnp.float32)
        + c1f_ref[...])
    h = lax.dot_general(nf, Wf_ref[...], (((1,), (1,)), ((), ())),
                        preferred_element_type=jnp.float32) + bf_ref[...]
    T_ref[0] = jnp.concatenate(
        [h, jnp.zeros_like(h)], axis=1)

    cnt = cnt_ref[0].sum(axis=0).reshape(-1, 1)
    hw = h * cnt
    S3 = hw.sum(axis=0).reshape(1, GR)
    Q3 = (hw * h).sum(axis=0).reshape(1, GR)
    S3Q3_ref[...] += jnp.concatenate([S3, Q3], axis=0)

    d = d_ref[0]
    D1_ref[...] += d.sum(axis=0).reshape(1, 4)
    D2_ref[...] += lax.dot_general(d, d, (((0,), (0,)), ((), ())),
                                   preferred_element_type=jnp.float32)


def _table_and_stats(feats, counts4, d_all, W1f, c1f, W_feats, b_feats):
    TN = 1024
    return pl.pallas_call(
        _c_body,
        grid=(B, N // TN),
        in_specs=[
            pl.BlockSpec((1, C_IN, TN), lambda b, t: (b, 0, t)),
            pl.BlockSpec((1, 4, TN), lambda b, t: (b, 0, t)),
            pl.BlockSpec((1, TN * K, 4), lambda b, t: (b, t, 0)),
            pl.BlockSpec((C_IN, C_IN), lambda b, t: (0, 0)),
            pl.BlockSpec((1, C_IN), lambda b, t: (0, 0)),
            pl.BlockSpec((GR, C_IN), lambda b, t: (0, 0)),
            pl.BlockSpec((1, GR), lambda b, t: (0, 0)),
        ],
        out_specs=[
            pl.BlockSpec((1, TN, 2 * GR), lambda b, t: (b, t, 0)),
            pl.BlockSpec((2, GR), lambda b, t: (0, 0)),
            pl.BlockSpec((1, 4), lambda b, t: (0, 0)),
            pl.BlockSpec((4, 4), lambda b, t: (0, 0)),
        ],
        out_shape=[
            jax.ShapeDtypeStruct((B, N, 2 * GR), jnp.float32),
            jax.ShapeDtypeStruct((2, GR), jnp.float32),
            jax.ShapeDtypeStruct((1, 4), jnp.float32),
            jax.ShapeDtypeStruct((4, 4), jnp.float32),
        ],
    )(feats, counts4, d_all, W1f, c1f, W_feats, b_feats)


def _d_body(is_final, G_ref, d_ref, a3_ref, c3_ref, A2_ref, c2_ref,
            Wp_ref, bp_ref, a4_ref, c4_ref, *refs):
    if is_final:
        out_refs = (refs[0], refs[1])
    else:
        out_refs = refs
    b = pl.program_id(0)
    t = pl.program_id(1)
    g = G_ref[0]
    f = jax.nn.relu(g * a3_ref[...] + c3_ref[...])
    d = d_ref[0]
    e = jax.nn.relu(
        lax.dot_general(d, A2_ref[...], (((1,), (0,)), ((), ())),
                        preferred_element_type=jnp.float32) + c2_ref[...])
    m = e * f
    p = lax.dot_general(m, Wp_ref[...], (((1,), (0,)), ((), ())),
                        preferred_element_type=jnp.float32) + bp_ref[...]
    if not is_final:
        sp_ref, spp_ref = out_refs

        @pl.when(jnp.logical_and(b == 0, t == 0))
        def _():
            sp_ref[...] = jnp.zeros_like(sp_ref)
            spp_ref[...] = jnp.zeros_like(spp_ref)

        sp_ref[...] += p.sum(axis=0).reshape(1, GR)
        spp_ref[...] += (p * p).sum(axis=0).reshape(1, GR)
    else:
        x_ref, out_ref = out_refs
        r = jax.nn.relu(p)
        TN2 = r.shape[0] // K
        nf = r.reshape(TN2, K, GR).sum(axis=1)
        out_ref[0] = jnp.concatenate(
            [x_ref[0], nf.T], axis=0)


def _d_pass(is_final, G, d_all, feats, a3, c3, A2p, c2, W_post, bp, a4, c4):
    TN2 = 1024
    TNK = TN2 * K
    nb = G.shape[0]
    in_specs = [
        pl.BlockSpec((1, TNK, 2 * GR), lambda b, t: (b, t, 0)),
        pl.BlockSpec((1, TNK, 4), lambda b, t: (b, t, 0)),
    ] + [pl.BlockSpec((1, 2 * GR), lambda b, t: (0, 0))] * 2       + [pl.BlockSpec((4, 2 * GR), lambda b, t: (0, 0)),
         pl.BlockSpec((1, 2 * GR), lambda b, t: (0, 0)),
         pl.BlockSpec((2 * GR, GR), lambda b, t: (0, 0))]       + [pl.BlockSpec((1, GR), lambda b, t: (0, 0))] * 3
    if is_final:
        in_specs.append(pl.BlockSpec((1, C_IN, TN2), lambda b, t: (b, 0, t)))
        out_specs = [pl.BlockSpec((1, C_IN + GR, TN2), lambda b, t: (b, 0, t))]
        out_shape = [jax.ShapeDtypeStruct((nb, C_IN + GR, N), jnp.float32)]
        args = (G, d_all, a3, c3, A2p, c2, W_post, bp, a4, c4, feats)
    else:
        out_specs = [pl.BlockSpec((1, GR), lambda b, t: (0, 0))] * 2
        out_shape = [jax.ShapeDtypeStruct((1, GR), jnp.float32)] * 2
        args = (G, d_all, a3, c3, A2p, c2, W_post, bp, a4, c4)
    res = pl.pallas_call(
        functools.partial(_d_body, is_final),
        grid=(nb, N // TN2),
        in_specs=in_specs,
        out_specs=out_specs,
        out_shape=out_shape,
    )(*args)
    return res


NW = 32
WPB = NW // B
RPW = NK // WPB
SC1_CH = 2048
SC2_CH = 256


def _wid():
    return lax.axis_index("s") * 2 + lax.axis_index("c")


def _sc1_body(idx_hbm, ptsp_hbm, cnt_hbm, d_hbm,
              idx_v, rows_v, pc_v, d_v, cnt_v, sem):
    w = _wid()
    b = w // WPB
    quarter = lax.rem(w, WPB)
    base = b * NK + quarter * RPW
    n0 = quarter * (N // WPB)
    ones = jnp.full((16,), 1.0, jnp.float32)
    zeros = jnp.zeros((16,), jnp.float32)
    iota = lax.iota(jnp.int32, 16)
    rowoff = iota // 4
    coloff = iota & 3

    def _zero(i, _):
        cnt_v[pl.ds(i * 16, 16)] = zeros
        return _
    lax.fori_loop(0, N // 16, _zero, None)

    def _chunk(ch, _):
        cbase = base + ch * SC1_CH
        pltpu.sync_copy(idx_hbm.at[pl.ds(cbase, SC1_CH)], idx_v)
        gat = pltpu.async_copy(ptsp_hbm.at[b].at[idx_v], rows_v, sem)

        def _hist(j, _):
            iv = idx_v[pl.ds(j * 16, 16)]
            plsc.addupdate_scatter(cnt_v, [iv], ones)
            return _
        lax.fori_loop(0, SC1_CH // 16, _hist, None)
        gat.wait()
        pltpu.sync_copy(
            ptsp_hbm.at[b].at[pl.ds(n0 + ch * (SC1_CH // K), SC1_CH // K)],
            pc_v)

        def _delta(n, _):
            pcv = plsc.load_gather(
                pc_v, [jnp.full((16,), n, jnp.int32), coloff])
            for j in range(4):
                rr = n * 16 + j * 4 + rowoff
                rv = plsc.load_gather(rows_v, [rr, coloff])
                plsc.store_scatter(d_v, [rr, coloff], rv - pcv)
            return _
        lax.fori_loop(0, SC1_CH // K, _delta, None)
        pltpu.sync_copy(d_v, d_hbm.at[b].at[pl.ds(quarter * RPW + ch * SC1_CH,
                                                  SC1_CH)])
        return _
    lax.fori_loop(0, RPW // SC1_CH, _chunk, None)
    pltpu.sync_copy(cnt_v, cnt_hbm.at[w])


def _sc1(knn_flat_1d, pts_pad16):
    mesh = plsc.VectorSubcoreMesh(core_axis_name="c", subcore_axis_name="s")
    return pl.kernel(
        _sc1_body,
        compiler_params=pltpu.CompilerParams(use_tc_tiling_on_sc=False, needs_layout_passes=False),
        out_type=[jax.ShapeDtypeStruct((NW, N), jnp.float32),
                  jax.ShapeDtypeStruct((B, NK, 4), jnp.float32)],
        mesh=mesh,
        scratch_types=[pltpu.VMEM((SC1_CH,), jnp.int32),
                       pltpu.VMEM((SC1_CH, 16), jnp.float32),
                       pltpu.VMEM((SC1_CH // K, 16), jnp.float32),
                       pltpu.VMEM((SC1_CH, 4), jnp.float32),
                       pltpu.VMEM((N,), jnp.float32),
                       pltpu.SemaphoreType.DMA],
    )(knn_flat_1d, pts_pad16)


def _sc2_body(nb, idx_hbm, T_hbm, G_hbm,
              idx_v0, idx_v1, rows_v0, rows_v1, sem0, sem1):
    w = _wid()
    wpb = NW // nb
    rpw = NK // wpb
    b = w // wpb
    part = lax.rem(w, wpb)
    base = b * NK + part * rpw
    lb = part * rpw
    nch = rpw // SC2_CH

    def _load(ch, idx_v, rows_v, sem):
        pltpu.sync_copy(idx_hbm.at[pl.ds(base + ch * SC2_CH, SC2_CH)], idx_v)
        pltpu.async_copy(T_hbm.at[b].at[idx_v], rows_v, sem)

    def _drain(idx_v, rows_v, sem):
        pltpu.make_async_copy(T_hbm.at[b].at[idx_v], rows_v, sem).wait()

    def _wb(ch, rows_v):
        pltpu.sync_copy(rows_v, G_hbm.at[b].at[pl.ds(lb + ch * SC2_CH,
                                                     SC2_CH)])

    _load(0, idx_v0, rows_v0, sem0)

    def _pair(j, _):
        _load(2 * j + 1, idx_v1, rows_v1, sem1)
        _drain(idx_v0, rows_v0, sem0)
        _wb(2 * j, rows_v0)

        @pl.when(j + 1 < nch // 2)
        def _():
            _load(2 * j + 2, idx_v0, rows_v0, sem0)
        _drain(idx_v1, rows_v1, sem1)
        _wb(2 * j + 1, rows_v1)
        return _
    lax.fori_loop(0, nch // 2, _pair, None)


def _sc2(knn_flat_1d, T):
    nb = T.shape[0]
    mesh = plsc.VectorSubcoreMesh(core_axis_name="c", subcore_axis_name="s")
    return pl.kernel(
        functools.partial(_sc2_body, nb),
        compiler_params=pltpu.CompilerParams(use_tc_tiling_on_sc=True, needs_layout_passes=False),
        out_type=jax.ShapeDtypeStruct((nb, NK, 2 * GR), jnp.float32),
        mesh=mesh,
        scratch_types=[pltpu.VMEM((SC2_CH,), jnp.int32),
                       pltpu.VMEM((SC2_CH,), jnp.int32),
                       pltpu.VMEM((SC2_CH, 2 * GR), jnp.float32),
                       pltpu.VMEM((SC2_CH, 2 * GR), jnp.float32),
                       pltpu.SemaphoreType.DMA,
                       pltpu.SemaphoreType.DMA],
    )(knn_flat_1d, T)


def kernel(feats, pts, knn_idx, W_bottle, b_bottle, g1, be1, W_delta, b_delta,
           g2, be2, W_feats, b_feats, g3, be3, W_post, b_post, g4, be4):
    knn_flat = knn_idx.reshape(B, NK).astype(jnp.int32)

    s1o, S1 = _feat_moments(feats)
    s1 = s1o[0]
    Ws = W_bottle @ s1
    sum_u = Ws + M1 * b_bottle
    sum_u2 = (jnp.einsum('oc,cd,od->o', W_bottle, S1, W_bottle)
              + 2 * b_bottle * Ws + M1 * b_bottle**2)
    mean1 = sum_u / M1
    var1 = sum_u2 / M1 - mean1**2
    a1 = g1 / jnp.sqrt(var1 + EPS)
    W1f = a1[:, None] * W_bottle
    c1f = (a1 * (b_bottle - mean1) + be1).reshape(1, C_IN)

    knn_1d = knn_flat.reshape(B * NK)
    pts_pad16 = jnp.concatenate(
        [pts.transpose(0, 2, 1), jnp.zeros((B, N, 13), jnp.float32)], axis=-1)
    counts32, d_all = _sc1(knn_1d, pts_pad16)
    counts4 = counts32.reshape(B, WPB, N)

    T, S3Q3, D1o, D2o = _table_and_stats(
        feats, counts4, d_all, W1f, c1f, W_feats, b_feats.reshape(1, GR))
    S3, Q3 = S3Q3[0], S3Q3[1]
    mean3 = S3 / M2
    var3 = Q3 / M2 - mean3**2
    a3 = (g3 / jnp.sqrt(var3 + EPS)).reshape(1, GR)
    c3 = (be3 - a3[0] * mean3).reshape(1, GR)

    D1 = D1o[0, :3]
    D2 = D2o[:3, :3]
    Wd1 = W_delta @ D1
    sum_z = Wd1 + M2 * b_delta
    sum_z2 = (jnp.einsum('oc,cd,od->o', W_delta, D2, W_delta)
              + 2 * b_delta * Wd1 + M2 * b_delta**2)
    mean2 = sum_z / M2
    var2 = sum_z2 / M2 - mean2**2
    a2 = g2 / jnp.sqrt(var2 + EPS)
    A2 = a2[:, None] * W_delta
    A2p = jnp.concatenate([A2.T, jnp.zeros((1, GR), jnp.float32)], axis=0)
    c2 = (a2 * (b_delta - mean2) + be2).reshape(1, GR)


    zz = jnp.zeros((1, GR), jnp.float32)
    z1 = jnp.zeros((1, GR), jnp.float32)
    a3p = jnp.concatenate([a3, z1], axis=1)
    c3p = jnp.concatenate([c3, z1], axis=1)
    A2pp = jnp.concatenate([A2p, jnp.zeros((4, GR), jnp.float32)], axis=1)
    c2p = jnp.concatenate([c2, z1], axis=1)
    Wp128 = jnp.concatenate([W_post.T, jnp.zeros((GR, GR), jnp.float32)],
                            axis=0)
    bp = b_post.reshape(1, GR)

    G = _sc2(knn_1d, T)
    sp, spp = _d_pass(False, G, d_all, None,
                      a3p, c3p, A2pp, c2p, Wp128, bp, zz, zz)
    mean4 = sp[0] / M2
    var4 = spp[0] / M2 - mean4**2
    a4 = (g4 / jnp.sqrt(var4 + EPS)).reshape(1, GR)
    c4 = (be4 - a4[0] * mean4).reshape(1, GR)

    Wp128f = Wp128 * a4
    bpf = (a4[0] * b_post + c4[0]).reshape(1, GR)
    (out,) = _d_pass(True, G, d_all, feats,
                     a3p, c3p, A2pp, c2p, Wp128f, bpf, a4, c4)
    return out
